# Initial kernel scaffold; baseline (speedup 1.0000x reference)
import jax
import jax.numpy as jnp
from jax.experimental import pallas as pl


def kernel(pts, nn_idx, nstep, W1, b1, W2, b2, Wt, Ws, bg):
    raise NotImplementedError("write your pallas kernel here")


# speedup vs baseline: 7.9557x; 7.9557x over previous
import functools

import jax
import jax.numpy as jnp
from jax import lax
from jax.experimental import pallas as pl
from jax.experimental.pallas import tpu as pltpu
from jax.experimental.pallas import tpu_sc as plsc

_B, _C, _N, _K, _A, _NT, _NOUT, _H = 1, 128, 10000, 16, 2, 8, 128, 32

_ROWS = _N * _K
_CHUNK = 128
_NW = 32
_ROWS_PAD = 163840
_CPW = _ROWS_PAD // (_NW * _CHUNK)

_BN = 400
_GRID = _N // _BN


def _sc_gather(table, idx):
    mesh = plsc.VectorSubcoreMesh(core_axis_name="c", subcore_axis_name="s")
    info = plsc.get_sparse_core_info()
    ncores = info.num_cores

    @functools.partial(
        pl.kernel,
        out_type=jax.ShapeDtypeStruct((_ROWS_PAD, _C), jnp.float32),
        mesh=mesh,
        scratch_types=[
            pltpu.VMEM((_CHUNK,), jnp.int32),
            pltpu.VMEM((_CHUNK, _C), jnp.float32),
            pltpu.SemaphoreType.DMA,
        ],
    )
    def gather_kernel(table_hbm, idx_hbm, out_hbm, idx_v, rows_v, sem):
        wid = lax.axis_index("s") * ncores + lax.axis_index("c")

        def body(c, carry):
            base = (wid * _CPW + c) * _CHUNK
            pltpu.sync_copy(idx_hbm.at[pl.ds(base, _CHUNK)], idx_v)
            pltpu.async_copy(table_hbm.at[idx_v], rows_v, sem).wait()
            pltpu.sync_copy(rows_v, out_hbm.at[pl.ds(base, _CHUNK)])
            return carry

        lax.fori_loop(0, _CPW, body, 0)

    return gather_kernel(table, idx)


def _tc_body(nbr_ref, ctr_ref, w1t_ref, b1_ref, w2t_ref, b2_ref, e_ref,
             wt2_ref, wst_ref, bg_ref, out_ref):
    ctr = ctr_ref[...]
    acc = jnp.zeros((_BN, _NT * _C), jnp.float32)
    for k in range(_K):
        nbr_k = nbr_ref[k]
        ef = nbr_k - ctr
        h = jnp.dot(ef, w1t_ref[...], preferred_element_type=jnp.float32)
        h = jnp.maximum(h + b1_ref[...], 0.0)
        lg = jnp.dot(h, w2t_ref[...], preferred_element_type=jnp.float32)
        lg = lg + b2_ref[...]
        m = jnp.max(lg, axis=1, keepdims=True)
        ex = jnp.exp(lg - m)
        et = ex / jnp.sum(ex, axis=1, keepdims=True)
        et_rep = jnp.dot(et, e_ref[...],
                         preferred_element_type=jnp.float32)
        nbr_tile = jnp.concatenate([nbr_k] * _NT, axis=1)
        acc = acc + et_rep * nbr_tile
    msg = jnp.dot(acc, wt2_ref[...], preferred_element_type=jnp.float32)
    self_t = jnp.dot(ctr, wst_ref[...], preferred_element_type=jnp.float32)
    out_ref[...] = jnp.maximum(msg + self_t + bg_ref[...], 0.0)


def _tc_call(nbr3, pts_t, w1t, b1r, w2t, b2r, e_mat, wt2, wst, bgr):
    return pl.pallas_call(
        _tc_body,
        grid=(_GRID,),
        in_specs=[
            pl.BlockSpec((_K, _BN, _C), lambda i: (0, i, 0)),
            pl.BlockSpec((_BN, _C), lambda i: (i, 0)),
            pl.BlockSpec((_C, _H), lambda i: (0, 0)),
            pl.BlockSpec((1, _H), lambda i: (0, 0)),
            pl.BlockSpec((_H, _NT), lambda i: (0, 0)),
            pl.BlockSpec((1, _NT), lambda i: (0, 0)),
            pl.BlockSpec((_NT, _NT * _C), lambda i: (0, 0)),
            pl.BlockSpec((_NT * _C, _NOUT), lambda i: (0, 0)),
            pl.BlockSpec((_C, _NOUT), lambda i: (0, 0)),
            pl.BlockSpec((1, _NOUT), lambda i: (0, 0)),
        ],
        out_specs=pl.BlockSpec((_BN, _NOUT), lambda i: (i, 0)),
        out_shape=jax.ShapeDtypeStruct((_N, _NOUT), jnp.float32),
    )(nbr3, pts_t, w1t, b1r, w2t, b2r, e_mat, wt2, wst, bgr)


def kernel(pts, nn_idx, nstep, W1, b1, W2, b2, Wt, Ws, bg):
    pts_t = pts[0].T
    idx = nn_idx[0].astype(jnp.int32).T.reshape(-1)
    idx = jnp.concatenate(
        [idx, jnp.zeros((_ROWS_PAD - _ROWS,), jnp.int32)])
    nbr = _sc_gather(pts_t, idx)
    nbr3 = nbr[:_ROWS].reshape(_K, _N, _C)

    mask = (jnp.asarray(nstep) == 0).astype(jnp.float32)
    w1t = W1.T
    w2t = W2.T
    b1r = b1.reshape(1, _H)
    b2r = b2.reshape(1, _NT)
    bgr = bg.reshape(1, _NOUT)
    e_mat = jnp.repeat(jnp.eye(_NT, dtype=jnp.float32), _C, axis=1)
    wt2 = (Wt * (mask / _K)).transpose(0, 2, 1).reshape(_NT * _C, _NOUT)
    wst = Ws.T

    y = _tc_call(nbr3, pts_t, w1t, b1r, w2t, b2r, e_mat, wt2, wst, bgr)
    out = jnp.broadcast_to(y.T[None, None, :, :, None],
                           (_B, _A, _NOUT, _N, 1))
    return out


# speedup vs baseline: 8.6787x; 1.0909x over previous
import functools

import jax
import jax.numpy as jnp
from jax import lax
from jax.experimental import pallas as pl
from jax.experimental.pallas import tpu as pltpu
from jax.experimental.pallas import tpu_sc as plsc

_B, _C, _N, _K, _A, _NT, _NOUT, _H = 1, 128, 10000, 16, 2, 8, 128, 32

_ROWS = _N * _K
_CHUNK = 128
_NW = 32
_ROWS_PAD = 163840
_CPW = _ROWS_PAD // (_NW * _CHUNK)
_NBUF = 4
_GROUPS = _CPW // _NBUF

_BN = 400
_GRID = _N // _BN


def _sc_gather(table, idx2):
    mesh = plsc.VectorSubcoreMesh(core_axis_name="c", subcore_axis_name="s")
    info = plsc.get_sparse_core_info()
    ncores = info.num_cores

    @functools.partial(
        pl.kernel,
        out_type=jax.ShapeDtypeStruct((_ROWS_PAD, _C), jnp.float32),
        mesh=mesh,
        scratch_types=[
            pltpu.VMEM((_CPW, _CHUNK), jnp.int32),
            pltpu.VMEM((_NBUF, _CHUNK, _C), jnp.float32),
            [pltpu.SemaphoreType.DMA] * _NBUF,
            [pltpu.SemaphoreType.DMA] * _NBUF,
        ],
    )
    def gather_kernel(table_hbm, idx_hbm, out_hbm, idx_all, rows_v,
                      gsems, ssems):
        wid = lax.axis_index("s") * ncores + lax.axis_index("c")
        pltpu.sync_copy(idx_hbm.at[pl.ds(wid * _CPW, _CPW)], idx_all)

        def wait_gather(b):
            pltpu.make_async_copy(
                table_hbm.at[pl.ds(0, _CHUNK)], rows_v.at[b],
                gsems[b]).wait()

        def wait_store(b):
            pltpu.make_async_copy(
                rows_v.at[b], out_hbm.at[pl.ds(0, _CHUNK)],
                ssems[b]).wait()

        @pl.loop(0, _GROUPS)
        def group(j):
            for b in range(_NBUF):
                c = j * _NBUF + b

                @pl.when(j > 0)
                def _():
                    wait_store(b)

                pltpu.async_copy(table_hbm.at[idx_all.at[c]],
                                 rows_v.at[b], gsems[b])
            for b in range(_NBUF):
                c = j * _NBUF + b
                wait_gather(b)
                base = (wid * _CPW + c) * _CHUNK
                pltpu.async_copy(rows_v.at[b],
                                 out_hbm.at[pl.ds(base, _CHUNK)], ssems[b])

        for b in range(_NBUF):
            wait_store(b)

    return gather_kernel(table, idx2)


def _tc_body(nbr_ref, ctr_ref, w1t_ref, b1_ref, w2t_ref, b2_ref, e_ref,
             wt2_ref, wst_ref, bg_ref, out_ref):
    ctr = ctr_ref[...]
    acc = jnp.zeros((_BN, _NT * _C), jnp.float32)
    for k in range(_K):
        nbr_k = nbr_ref[k]
        ef = nbr_k - ctr
        h = jnp.dot(ef, w1t_ref[...], preferred_element_type=jnp.float32)
        h = jnp.maximum(h + b1_ref[...], 0.0)
        lg = jnp.dot(h, w2t_ref[...], preferred_element_type=jnp.float32)
        lg = lg + b2_ref[...]
        m = jnp.max(lg, axis=1, keepdims=True)
        ex = jnp.exp(lg - m)
        et = ex / jnp.sum(ex, axis=1, keepdims=True)
        et_rep = jnp.dot(et, e_ref[...],
                         preferred_element_type=jnp.float32)
        nbr_tile = jnp.concatenate([nbr_k] * _NT, axis=1)
        acc = acc + et_rep * nbr_tile
    msg = jnp.dot(acc, wt2_ref[...], preferred_element_type=jnp.float32)
    self_t = jnp.dot(ctr, wst_ref[...], preferred_element_type=jnp.float32)
    out_ref[...] = jnp.maximum(msg + self_t + bg_ref[...], 0.0)


def _tc_call(nbr3, pts_t, w1t, b1r, w2t, b2r, e_mat, wt2, wst, bgr):
    return pl.pallas_call(
        _tc_body,
        grid=(_GRID,),
        in_specs=[
            pl.BlockSpec((_K, _BN, _C), lambda i: (0, i, 0)),
            pl.BlockSpec((_BN, _C), lambda i: (i, 0)),
            pl.BlockSpec((_C, _H), lambda i: (0, 0)),
            pl.BlockSpec((1, _H), lambda i: (0, 0)),
            pl.BlockSpec((_H, _NT), lambda i: (0, 0)),
            pl.BlockSpec((1, _NT), lambda i: (0, 0)),
            pl.BlockSpec((_NT, _NT * _C), lambda i: (0, 0)),
            pl.BlockSpec((_NT * _C, _NOUT), lambda i: (0, 0)),
            pl.BlockSpec((_C, _NOUT), lambda i: (0, 0)),
            pl.BlockSpec((1, _NOUT), lambda i: (0, 0)),
        ],
        out_specs=pl.BlockSpec((_BN, _NOUT), lambda i: (i, 0)),
        out_shape=jax.ShapeDtypeStruct((_N, _NOUT), jnp.float32),
    )(nbr3, pts_t, w1t, b1r, w2t, b2r, e_mat, wt2, wst, bgr)


def kernel(pts, nn_idx, nstep, W1, b1, W2, b2, Wt, Ws, bg):
    pts_t = pts[0].T
    idx = nn_idx[0].astype(jnp.int32).T.reshape(-1)
    idx2 = jnp.concatenate(
        [idx, jnp.zeros((_ROWS_PAD - _ROWS,), jnp.int32)]).reshape(
            _ROWS_PAD // _CHUNK, _CHUNK)
    nbr = _sc_gather(pts_t, idx2)
    nbr3 = nbr[:_ROWS].reshape(_K, _N, _C)

    mask = (jnp.asarray(nstep) == 0).astype(jnp.float32)
    w1t = W1.T
    w2t = W2.T
    b1r = b1.reshape(1, _H)
    b2r = b2.reshape(1, _NT)
    bgr = bg.reshape(1, _NOUT)
    e_mat = jnp.repeat(jnp.eye(_NT, dtype=jnp.float32), _C, axis=1)
    wt2 = (Wt * (mask / _K)).transpose(0, 2, 1).reshape(_NT * _C, _NOUT)
    wst = Ws.T

    y = _tc_call(nbr3, pts_t, w1t, b1r, w2t, b2r, e_mat, wt2, wst, bgr)
    out = jnp.broadcast_to(y.T[None, None, :, :, None],
                           (_B, _A, _NOUT, _N, 1))
    return out
